# baseline (speedup 1.0000x reference)
import functools

import jax
import jax.numpy as jnp
from jax import lax
from jax.experimental import pallas as pl
from jax.experimental.pallas import tpu as pltpu
from jax.experimental.pallas import tpu_sc as plsc

N_ROWS = 160000
N_SEGMENTS = 10000
D_IN = 128
NUM_RF = 256
D_OUT = 128


ROW_BLOCK = 1280

_INV_SQRT_2PI = 0.3989422804014327
_SQRT_2_OVER_PI = 0.7978845608028654


def _vb_body(x_ref, w0m_ref, w0lv_ref, b0m_ref, b0lv_ref,
             w1m_ref, w1lv_ref, b1m_ref, b1lv_ref,
             prec_ref, wtd_ref, w0v_s, w1v_s, w1vpm_s):
    @pl.when(pl.program_id(0) == 0)
    def _():
        w0v_s[...] = jnp.exp(w0lv_ref[...])
        w1v = jnp.exp(w1lv_ref[...])
        w1v_s[...] = w1v
        w1m = w1m_ref[...]
        w1vpm_s[...] = w1v + w1m * w1m

    x = x_ref[...]
    b0m = b0m_ref[...]
    b0v = jnp.exp(b0lv_ref[...])

    out_mean = jnp.dot(x, w0m_ref[...], preferred_element_type=jnp.float32) + b0m
    out_var = jnp.dot(x * x, w0v_s[...], preferred_element_type=jnp.float32) + b0v
    out_var = jnp.maximum(out_var, 1e-8)

    r = lax.rsqrt(out_var)
    a = out_mean * r
    s = out_var * r
    a2 = a * a
    cdf = 0.5 + 0.5 * jnp.tanh(_SQRT_2_OVER_PI * (a * (1.0 + 0.044715 * a2)))
    pdf = jnp.exp(-0.5 * a2) * _INV_SQRT_2PI
    m1 = out_mean * cdf + s * pdf
    v1 = m1 * (out_mean - m1) + out_var * cdf
    v1 = jnp.maximum(v1, 1e-8)

    b1m = b1m_ref[...]
    b1v = jnp.exp(b1lv_ref[...])

    m2 = jnp.dot(m1, w1m_ref[...], preferred_element_type=jnp.float32) + b1m
    v2 = (jnp.dot(v1, w1vpm_s[...], preferred_element_type=jnp.float32)
          + jnp.dot(m1 * m1, w1v_s[...], preferred_element_type=jnp.float32)
          + b1v)
    v2 = jnp.maximum(v2, 1e-8)

    prec = 1.0 / (v2 + 1e-8)
    prec_ref[...] = prec
    wtd_ref[...] = prec * m2


def _vb_stage(X, W0_mean, W0_logvar, b0_mean, b0_logvar,
              W1_mean, W1_logvar, b1_mean, b1_logvar):
    n_rows = X.shape[0]
    full = lambda i: (0, 0)
    row = lambda i: (i, 0)
    return pl.pallas_call(
        _vb_body,
        grid=(n_rows // ROW_BLOCK,),
        in_specs=[
            pl.BlockSpec((ROW_BLOCK, D_IN), row),
            pl.BlockSpec((D_IN, NUM_RF), full),
            pl.BlockSpec((D_IN, NUM_RF), full),
            pl.BlockSpec((NUM_RF,), lambda i: (0,)),
            pl.BlockSpec((NUM_RF,), lambda i: (0,)),
            pl.BlockSpec((NUM_RF, D_OUT), full),
            pl.BlockSpec((NUM_RF, D_OUT), full),
            pl.BlockSpec((D_OUT,), lambda i: (0,)),
            pl.BlockSpec((D_OUT,), lambda i: (0,)),
        ],
        out_specs=[
            pl.BlockSpec((ROW_BLOCK, D_OUT), row),
            pl.BlockSpec((ROW_BLOCK, D_OUT), row),
        ],
        out_shape=[
            jax.ShapeDtypeStruct((n_rows, D_OUT), jnp.float32),
            jax.ShapeDtypeStruct((n_rows, D_OUT), jnp.float32),
        ],
        scratch_shapes=[
            pltpu.VMEM((D_IN, NUM_RF), jnp.float32),
            pltpu.VMEM((NUM_RF, D_OUT), jnp.float32),
            pltpu.VMEM((NUM_RF, D_OUT), jnp.float32),
        ],
    )(X, W0_mean, W0_logvar, b0_mean, b0_logvar,
      W1_mean, W1_logvar, b1_mean, b1_logvar)


NUM_TILES = 16
CHUNK = 128
N_CHUNKS = N_ROWS // CHUNK
SEG_PAD = 10240
SEG_PER_TILE = SEG_PAD // NUM_TILES


def _seg_tile_work(src_hbm, idx_v, acc_sh, buf0, buf1, sem0, sem1, s,
                   n_chunks, cpt):
    base = s * cpt

    def start_load(k, buf, sem):
        @pl.when((k < cpt) & (base + k < n_chunks))
        def _():
            pltpu.async_copy(src_hbm.at[pl.ds((base + k) * CHUNK, CHUNK)],
                             buf, sem)

    def finish(k, buf, sem):
        @pl.when(base + k < n_chunks)
        def _():
            pltpu.make_async_copy(src_hbm.at[pl.ds((base + k) * CHUNK, CHUNK)],
                                  buf, sem).wait()
            pltpu.sync_copy(buf, acc_sh.at[idx_v.at[k]], add=True)

    start_load(0, buf0, sem0)
    start_load(1, buf1, sem1)

    def outer(j, _):
        k = j * 2
        finish(k, buf0, sem0)
        start_load(k + 2, buf0, sem0)
        finish(k + 1, buf1, sem1)
        start_load(k + 3, buf1, sem1)
        return 0
    lax.fori_loop(0, cpt // 2, outer, 0)


def _seg_body(n_chunks, cpt,
              prec_hbm, wtd_hbm, idx_hbm, zeros_hbm,
              wsum_hbm, msum_hbm,
              buf0, buf1, idx_v, acc_sh, sem0, sem1):
    c = lax.axis_index("c")
    s = lax.axis_index("s")
    seg0 = s * SEG_PER_TILE
    pltpu.sync_copy(zeros_hbm.at[pl.ds(seg0, SEG_PER_TILE)],
                    acc_sh.at[pl.ds(seg0, SEG_PER_TILE)])
    pltpu.sync_copy(idx_hbm.at[pl.ds(s * cpt, cpt)], idx_v)
    plsc.subcore_barrier()

    @pl.when(c == 0)
    def _():
        _seg_tile_work(prec_hbm, idx_v, acc_sh, buf0, buf1, sem0, sem1, s,
                       n_chunks, cpt)

    @pl.when(c == 1)
    def _():
        _seg_tile_work(wtd_hbm, idx_v, acc_sh, buf0, buf1, sem0, sem1, s,
                       n_chunks, cpt)

    plsc.subcore_barrier()

    @pl.when(c == 0)
    def _():
        pltpu.sync_copy(acc_sh.at[pl.ds(seg0, SEG_PER_TILE)],
                        wsum_hbm.at[pl.ds(seg0, SEG_PER_TILE)])

    @pl.when(c == 1)
    def _():
        pltpu.sync_copy(acc_sh.at[pl.ds(seg0, SEG_PER_TILE)],
                        msum_hbm.at[pl.ds(seg0, SEG_PER_TILE)])


def _segment_sums(precision, weighted, idx2d, zeros):
    n_chunks = precision.shape[0] // CHUNK
    cpt = idx2d.shape[0] // NUM_TILES
    mesh = plsc.VectorSubcoreMesh(core_axis_name="c", subcore_axis_name="s")
    f = pl.kernel(
        functools.partial(_seg_body, n_chunks, cpt),
        out_type=(
            jax.ShapeDtypeStruct((SEG_PAD, D_OUT), jnp.float32),
            jax.ShapeDtypeStruct((SEG_PAD, D_OUT), jnp.float32),
        ),
        mesh=mesh,
        scratch_types=[
            pltpu.VMEM((CHUNK, D_OUT), jnp.float32),
            pltpu.VMEM((CHUNK, D_OUT), jnp.float32),
            pltpu.VMEM((cpt, CHUNK), jnp.int32),
            pltpu.VMEM_SHARED((SEG_PAD, D_OUT), jnp.float32),
            pltpu.SemaphoreType.DMA,
            pltpu.SemaphoreType.DMA,
        ],
    )
    return f(precision, weighted, idx2d, zeros)


SEG_BLOCK = 2000


def _norm_body(wa_ref, wb_ref, ma_ref, mb_ref, mean_ref, var_ref):
    w = wa_ref[...] + wb_ref[...] + 1e-8
    vi = 1.0 / w
    var_ref[...] = vi
    mean_ref[...] = (ma_ref[...] + mb_ref[...]) * vi


def _normalize(w_a, w_b, m_a, m_b):
    row = lambda i: (i, 0)
    spec = pl.BlockSpec((SEG_BLOCK, D_OUT), row)
    return pl.pallas_call(
        _norm_body,
        grid=(N_SEGMENTS // SEG_BLOCK,),
        in_specs=[spec, spec, spec, spec],
        out_specs=[spec, spec],
        out_shape=[jax.ShapeDtypeStruct((N_SEGMENTS, D_OUT), jnp.float32),
                   jax.ShapeDtypeStruct((N_SEGMENTS, D_OUT), jnp.float32)],
    )(w_a, w_b, m_a, m_b)


ROWS_H = 81920
CHUNKS_H = ROWS_H // CHUNK
CPT_H = 40
IDX_PAD_H = NUM_TILES * CPT_H


def kernel(X, X_idx, W0_mean, W0_logvar, b0_mean, b0_logvar,
           W1_mean, W1_logvar, b1_mean, b1_logvar):
    idx_all = X_idx.reshape(N_CHUNKS, CHUNK)
    idx_a = idx_all[:CHUNKS_H]
    pad = jnp.zeros((IDX_PAD_H - (N_CHUNKS - CHUNKS_H), CHUNK), jnp.int32)
    idx_b = jnp.concatenate([idx_all[CHUNKS_H:], pad], axis=0)
    zeros = jnp.zeros((SEG_PAD, D_OUT), jnp.float32)

    prec_a, wtd_a = _vb_stage(
        X[:ROWS_H], W0_mean, W0_logvar, b0_mean, b0_logvar,
        W1_mean, W1_logvar, b1_mean, b1_logvar)
    w_a, m_a = _segment_sums(prec_a, wtd_a, idx_a, zeros)
    prec_b, wtd_b = _vb_stage(
        X[ROWS_H:], W0_mean, W0_logvar, b0_mean, b0_logvar,
        W1_mean, W1_logvar, b1_mean, b1_logvar)
    w_b, m_b = _segment_sums(prec_b, wtd_b, idx_b, zeros)

    embedd_means, embedd_vars = _normalize(
        w_a[:N_SEGMENTS], w_b[:N_SEGMENTS], m_a[:N_SEGMENTS], m_b[:N_SEGMENTS])
    return (embedd_means, embedd_vars)


# speedup vs baseline: 1.1745x; 1.1745x over previous
import functools

import jax
import jax.numpy as jnp
from jax import lax
from jax.experimental import pallas as pl
from jax.experimental.pallas import tpu as pltpu
from jax.experimental.pallas import tpu_sc as plsc

N_ROWS = 160000
N_SEGMENTS = 10000
D_IN = 128
NUM_RF = 256
D_OUT = 128


ROW_BLOCK = 1280

_INV_SQRT_2PI = 0.3989422804014327
_SQRT_2_OVER_PI = 0.7978845608028654


def _vb_body(x_ref, w0m_ref, w0lv_ref, b0m_ref, b0lv_ref,
             w1m_ref, w1lv_ref, b1m_ref, b1lv_ref,
             prec_ref, wtd_ref, w0v_s, w1v_s, w1vpm_s):
    @pl.when(pl.program_id(0) == 0)
    def _():
        w0v_s[...] = jnp.exp(w0lv_ref[...])
        w1v = jnp.exp(w1lv_ref[...])
        w1v_s[...] = w1v
        w1m = w1m_ref[...]
        w1vpm_s[...] = w1v + w1m * w1m

    x = x_ref[...]
    b0m = b0m_ref[...]
    b0v = jnp.exp(b0lv_ref[...])

    out_mean = jnp.dot(x, w0m_ref[...], preferred_element_type=jnp.float32) + b0m
    out_var = jnp.dot(x * x, w0v_s[...], preferred_element_type=jnp.float32) + b0v
    out_var = jnp.maximum(out_var, 1e-8)

    r = lax.rsqrt(out_var)
    a = out_mean * r
    s = out_var * r
    a2 = a * a
    cdf = 0.5 + 0.5 * jnp.tanh(_SQRT_2_OVER_PI * (a * (1.0 + 0.044715 * a2)))
    pdf = jnp.exp(-0.5 * a2) * _INV_SQRT_2PI
    m1 = out_mean * cdf + s * pdf
    v1 = m1 * (out_mean - m1) + out_var * cdf
    v1 = jnp.maximum(v1, 1e-8)

    b1m = b1m_ref[...]
    b1v = jnp.exp(b1lv_ref[...])

    m2 = jnp.dot(m1, w1m_ref[...], preferred_element_type=jnp.float32) + b1m
    v2 = (jnp.dot(v1, w1vpm_s[...], preferred_element_type=jnp.float32)
          + jnp.dot(m1 * m1, w1v_s[...], preferred_element_type=jnp.float32)
          + b1v)
    v2 = jnp.maximum(v2, 1e-8)

    prec = 1.0 / (v2 + 1e-8)
    prec_ref[...] = prec
    wtd_ref[...] = prec * m2


def _vb_stage(X, block_off, n_blocks,
              W0_mean, W0_logvar, b0_mean, b0_logvar,
              W1_mean, W1_logvar, b1_mean, b1_logvar):
    n_rows = n_blocks * ROW_BLOCK
    full = lambda i: (0, 0)
    row = lambda i: (i + block_off, 0)
    out_row = lambda i: (i, 0)
    return pl.pallas_call(
        _vb_body,
        grid=(n_blocks,),
        in_specs=[
            pl.BlockSpec((ROW_BLOCK, D_IN), row),
            pl.BlockSpec((D_IN, NUM_RF), full),
            pl.BlockSpec((D_IN, NUM_RF), full),
            pl.BlockSpec((NUM_RF,), lambda i: (0,)),
            pl.BlockSpec((NUM_RF,), lambda i: (0,)),
            pl.BlockSpec((NUM_RF, D_OUT), full),
            pl.BlockSpec((NUM_RF, D_OUT), full),
            pl.BlockSpec((D_OUT,), lambda i: (0,)),
            pl.BlockSpec((D_OUT,), lambda i: (0,)),
        ],
        out_specs=[
            pl.BlockSpec((ROW_BLOCK, D_OUT), out_row),
            pl.BlockSpec((ROW_BLOCK, D_OUT), out_row),
        ],
        out_shape=[
            jax.ShapeDtypeStruct((n_rows, D_OUT), jnp.float32),
            jax.ShapeDtypeStruct((n_rows, D_OUT), jnp.float32),
        ],
        scratch_shapes=[
            pltpu.VMEM((D_IN, NUM_RF), jnp.float32),
            pltpu.VMEM((NUM_RF, D_OUT), jnp.float32),
            pltpu.VMEM((NUM_RF, D_OUT), jnp.float32),
        ],
    )(X, W0_mean, W0_logvar, b0_mean, b0_logvar,
      W1_mean, W1_logvar, b1_mean, b1_logvar)


NUM_TILES = 16
CHUNK = 128
N_CHUNKS = N_ROWS // CHUNK
SEG_PAD = 10240
SEG_PER_TILE = SEG_PAD // NUM_TILES


def _seg_tile_work(src_hbm, idx_v, acc_sh, buf0, buf1, sem0, sem1, s,
                   n_chunks, cpt):
    base = s * cpt

    def start_load(k, buf, sem):
        @pl.when((k < cpt) & (base + k < n_chunks))
        def _():
            pltpu.async_copy(src_hbm.at[pl.ds((base + k) * CHUNK, CHUNK)],
                             buf, sem)

    def finish(k, buf, sem):
        @pl.when(base + k < n_chunks)
        def _():
            pltpu.make_async_copy(src_hbm.at[pl.ds((base + k) * CHUNK, CHUNK)],
                                  buf, sem).wait()
            pltpu.sync_copy(buf, acc_sh.at[idx_v.at[k]], add=True)

    start_load(0, buf0, sem0)
    start_load(1, buf1, sem1)

    def outer(j, _):
        k = j * 2
        finish(k, buf0, sem0)
        start_load(k + 2, buf0, sem0)
        finish(k + 1, buf1, sem1)
        start_load(k + 3, buf1, sem1)
        return 0
    lax.fori_loop(0, cpt // 2, outer, 0)


def _seg_body(n_chunks, cpt,
              prec_hbm, wtd_hbm, idx_hbm, zeros_hbm,
              wsum_hbm, msum_hbm,
              buf0, buf1, idx_v, acc_sh, sem0, sem1):
    c = lax.axis_index("c")
    s = lax.axis_index("s")
    seg0 = s * SEG_PER_TILE
    pltpu.sync_copy(zeros_hbm.at[pl.ds(seg0, SEG_PER_TILE)],
                    acc_sh.at[pl.ds(seg0, SEG_PER_TILE)])
    pltpu.sync_copy(idx_hbm.at[pl.ds(s * cpt, cpt)], idx_v)
    plsc.subcore_barrier()

    @pl.when(c == 0)
    def _():
        _seg_tile_work(prec_hbm, idx_v, acc_sh, buf0, buf1, sem0, sem1, s,
                       n_chunks, cpt)

    @pl.when(c == 1)
    def _():
        _seg_tile_work(wtd_hbm, idx_v, acc_sh, buf0, buf1, sem0, sem1, s,
                       n_chunks, cpt)

    plsc.subcore_barrier()

    @pl.when(c == 0)
    def _():
        pltpu.sync_copy(acc_sh.at[pl.ds(seg0, SEG_PER_TILE)],
                        wsum_hbm.at[pl.ds(seg0, SEG_PER_TILE)])

    @pl.when(c == 1)
    def _():
        pltpu.sync_copy(acc_sh.at[pl.ds(seg0, SEG_PER_TILE)],
                        msum_hbm.at[pl.ds(seg0, SEG_PER_TILE)])


def _segment_sums(precision, weighted, idx2d, zeros):
    n_chunks = precision.shape[0] // CHUNK
    cpt = idx2d.shape[0] // NUM_TILES
    mesh = plsc.VectorSubcoreMesh(core_axis_name="c", subcore_axis_name="s")
    f = pl.kernel(
        functools.partial(_seg_body, n_chunks, cpt),
        out_type=(
            jax.ShapeDtypeStruct((SEG_PAD, D_OUT), jnp.float32),
            jax.ShapeDtypeStruct((SEG_PAD, D_OUT), jnp.float32),
        ),
        mesh=mesh,
        scratch_types=[
            pltpu.VMEM((CHUNK, D_OUT), jnp.float32),
            pltpu.VMEM((CHUNK, D_OUT), jnp.float32),
            pltpu.VMEM((cpt, CHUNK), jnp.int32),
            pltpu.VMEM_SHARED((SEG_PAD, D_OUT), jnp.float32),
            pltpu.SemaphoreType.DMA,
            pltpu.SemaphoreType.DMA,
        ],
    )
    return f(precision, weighted, idx2d, zeros)


SEG_BLOCK = 2000


def _norm_body(wa_ref, wb_ref, ma_ref, mb_ref, mean_ref, var_ref):
    w = wa_ref[...] + wb_ref[...] + 1e-8
    vi = 1.0 / w
    var_ref[...] = vi
    mean_ref[...] = (ma_ref[...] + mb_ref[...]) * vi


def _normalize(w_a, w_b, m_a, m_b):
    row = lambda i: (i, 0)
    spec = pl.BlockSpec((SEG_BLOCK, D_OUT), row)
    return pl.pallas_call(
        _norm_body,
        grid=(N_SEGMENTS // SEG_BLOCK,),
        in_specs=[spec, spec, spec, spec],
        out_specs=[spec, spec],
        out_shape=[jax.ShapeDtypeStruct((N_SEGMENTS, D_OUT), jnp.float32),
                   jax.ShapeDtypeStruct((N_SEGMENTS, D_OUT), jnp.float32)],
    )(w_a, w_b, m_a, m_b)


ROWS_H = 81920
CHUNKS_H = ROWS_H // CHUNK
CPT_H = 40
IDX_PAD_H = NUM_TILES * CPT_H


def kernel(X, X_idx, W0_mean, W0_logvar, b0_mean, b0_logvar,
           W1_mean, W1_logvar, b1_mean, b1_logvar):
    idx_all = X_idx.reshape(N_CHUNKS, CHUNK)
    idx_a = idx_all[:CHUNKS_H]
    pad = jnp.zeros((IDX_PAD_H - (N_CHUNKS - CHUNKS_H), CHUNK), jnp.int32)
    idx_b = jnp.concatenate([idx_all[CHUNKS_H:], pad], axis=0)
    zeros = jnp.zeros((SEG_PAD, D_OUT), jnp.float32)

    blocks_a = ROWS_H // ROW_BLOCK
    blocks_b = (N_ROWS - ROWS_H) // ROW_BLOCK
    prec_a, wtd_a = _vb_stage(
        X, 0, blocks_a, W0_mean, W0_logvar, b0_mean, b0_logvar,
        W1_mean, W1_logvar, b1_mean, b1_logvar)
    w_a, m_a = _segment_sums(prec_a, wtd_a, idx_a, zeros)
    prec_b, wtd_b = _vb_stage(
        X, blocks_a, blocks_b, W0_mean, W0_logvar, b0_mean, b0_logvar,
        W1_mean, W1_logvar, b1_mean, b1_logvar)
    w_b, m_b = _segment_sums(prec_b, wtd_b, idx_b, zeros)

    embedd_means, embedd_vars = _normalize(
        w_a[:N_SEGMENTS], w_b[:N_SEGMENTS], m_a[:N_SEGMENTS], m_b[:N_SEGMENTS])
    return (embedd_means, embedd_vars)


# speedup vs baseline: 1.2773x; 1.0874x over previous
import functools

import jax
import jax.numpy as jnp
from jax import lax
from jax.experimental import pallas as pl
from jax.experimental.pallas import tpu as pltpu
from jax.experimental.pallas import tpu_sc as plsc

N_ROWS = 160000
N_SEGMENTS = 10000
D_IN = 128
NUM_RF = 256
D_OUT = 128


ROW_BLOCK = 2000

_INV_SQRT_2PI = 0.3989422804014327
_SQRT_2_OVER_PI = 0.7978845608028654


def _vb_body(x_ref, w0m_ref, w0lv_ref, b0m_ref, b0lv_ref,
             w1m_ref, w1lv_ref, b1m_ref, b1lv_ref,
             prec_ref, wtd_ref, w0v_s, w1v_s, w1vpm_s):
    @pl.when(pl.program_id(0) == 0)
    def _():
        w0v_s[...] = jnp.exp(w0lv_ref[...])
        w1v = jnp.exp(w1lv_ref[...])
        w1v_s[...] = w1v
        w1m = w1m_ref[...]
        w1vpm_s[...] = w1v + w1m * w1m

    x = x_ref[...]
    b0m = b0m_ref[...]
    b0v = jnp.exp(b0lv_ref[...])

    out_mean = jnp.dot(x, w0m_ref[...], preferred_element_type=jnp.float32) + b0m
    out_var = jnp.dot(x * x, w0v_s[...], preferred_element_type=jnp.float32) + b0v
    out_var = jnp.maximum(out_var, 1e-8)

    r = lax.rsqrt(out_var)
    a = out_mean * r
    s = out_var * r
    a2 = a * a
    cdf = 0.5 + 0.5 * jnp.tanh(_SQRT_2_OVER_PI * (a * (1.0 + 0.044715 * a2)))
    pdf = jnp.exp(-0.5 * a2) * _INV_SQRT_2PI
    m1 = out_mean * cdf + s * pdf
    v1 = m1 * (out_mean - m1) + out_var * cdf
    v1 = jnp.maximum(v1, 1e-8)

    b1m = b1m_ref[...]
    b1v = jnp.exp(b1lv_ref[...])

    m2 = jnp.dot(m1, w1m_ref[...], preferred_element_type=jnp.float32) + b1m
    v2 = (jnp.dot(v1, w1vpm_s[...], preferred_element_type=jnp.float32)
          + jnp.dot(m1 * m1, w1v_s[...], preferred_element_type=jnp.float32)
          + b1v)
    v2 = jnp.maximum(v2, 1e-8)

    prec = 1.0 / (v2 + 1e-8)
    prec_ref[...] = prec
    wtd_ref[...] = prec * m2


def _vb_stage(X, block_off, n_blocks,
              W0_mean, W0_logvar, b0_mean, b0_logvar,
              W1_mean, W1_logvar, b1_mean, b1_logvar):
    n_rows = n_blocks * ROW_BLOCK
    full = lambda i: (0, 0)
    row = lambda i: (i + block_off, 0)
    out_row = lambda i: (i, 0)
    return pl.pallas_call(
        _vb_body,
        grid=(n_blocks,),
        in_specs=[
            pl.BlockSpec((ROW_BLOCK, D_IN), row),
            pl.BlockSpec((D_IN, NUM_RF), full),
            pl.BlockSpec((D_IN, NUM_RF), full),
            pl.BlockSpec((NUM_RF,), lambda i: (0,)),
            pl.BlockSpec((NUM_RF,), lambda i: (0,)),
            pl.BlockSpec((NUM_RF, D_OUT), full),
            pl.BlockSpec((NUM_RF, D_OUT), full),
            pl.BlockSpec((D_OUT,), lambda i: (0,)),
            pl.BlockSpec((D_OUT,), lambda i: (0,)),
        ],
        out_specs=[
            pl.BlockSpec((ROW_BLOCK, D_OUT), out_row),
            pl.BlockSpec((ROW_BLOCK, D_OUT), out_row),
        ],
        out_shape=[
            jax.ShapeDtypeStruct((n_rows, D_OUT), jnp.float32),
            jax.ShapeDtypeStruct((n_rows, D_OUT), jnp.float32),
        ],
        scratch_shapes=[
            pltpu.VMEM((D_IN, NUM_RF), jnp.float32),
            pltpu.VMEM((NUM_RF, D_OUT), jnp.float32),
            pltpu.VMEM((NUM_RF, D_OUT), jnp.float32),
        ],
    )(X, W0_mean, W0_logvar, b0_mean, b0_logvar,
      W1_mean, W1_logvar, b1_mean, b1_logvar)


NUM_TILES = 16
CHUNK = 128
N_CHUNKS = N_ROWS // CHUNK
SEG_PAD = 10240
SEG_PER_TILE = SEG_PAD // NUM_TILES


def _seg_tile_work(src_hbm, idx_v, acc_sh, buf0, buf1, sem0, sem1, s,
                   n_chunks, cpt):
    base = s * cpt

    def start_load(k, buf, sem):
        @pl.when((k < cpt) & (base + k < n_chunks))
        def _():
            pltpu.async_copy(src_hbm.at[pl.ds((base + k) * CHUNK, CHUNK)],
                             buf, sem)

    def finish(k, buf, sem):
        @pl.when(base + k < n_chunks)
        def _():
            pltpu.make_async_copy(src_hbm.at[pl.ds((base + k) * CHUNK, CHUNK)],
                                  buf, sem).wait()
            pltpu.sync_copy(buf, acc_sh.at[idx_v.at[k]], add=True)

    start_load(0, buf0, sem0)
    start_load(1, buf1, sem1)

    def outer(j, _):
        k = j * 2
        finish(k, buf0, sem0)
        start_load(k + 2, buf0, sem0)
        finish(k + 1, buf1, sem1)
        start_load(k + 3, buf1, sem1)
        return 0
    lax.fori_loop(0, cpt // 2, outer, 0)


def _seg_body(n_chunks, cpt,
              prec_hbm, wtd_hbm, idx_hbm, zeros_hbm,
              wsum_hbm, msum_hbm,
              buf0, buf1, idx_v, acc_sh, sem0, sem1):
    c = lax.axis_index("c")
    s = lax.axis_index("s")
    seg0 = s * SEG_PER_TILE
    pltpu.sync_copy(zeros_hbm.at[pl.ds(seg0, SEG_PER_TILE)],
                    acc_sh.at[pl.ds(seg0, SEG_PER_TILE)])
    pltpu.sync_copy(idx_hbm.at[pl.ds(s * cpt, cpt)], idx_v)
    plsc.subcore_barrier()

    @pl.when(c == 0)
    def _():
        _seg_tile_work(prec_hbm, idx_v, acc_sh, buf0, buf1, sem0, sem1, s,
                       n_chunks, cpt)

    @pl.when(c == 1)
    def _():
        _seg_tile_work(wtd_hbm, idx_v, acc_sh, buf0, buf1, sem0, sem1, s,
                       n_chunks, cpt)

    plsc.subcore_barrier()

    @pl.when(c == 0)
    def _():
        pltpu.sync_copy(acc_sh.at[pl.ds(seg0, SEG_PER_TILE)],
                        wsum_hbm.at[pl.ds(seg0, SEG_PER_TILE)])

    @pl.when(c == 1)
    def _():
        pltpu.sync_copy(acc_sh.at[pl.ds(seg0, SEG_PER_TILE)],
                        msum_hbm.at[pl.ds(seg0, SEG_PER_TILE)])


def _segment_sums(precision, weighted, idx2d, zeros):
    n_chunks = precision.shape[0] // CHUNK
    cpt = idx2d.shape[0] // NUM_TILES
    mesh = plsc.VectorSubcoreMesh(core_axis_name="c", subcore_axis_name="s")
    f = pl.kernel(
        functools.partial(_seg_body, n_chunks, cpt),
        out_type=(
            jax.ShapeDtypeStruct((SEG_PAD, D_OUT), jnp.float32),
            jax.ShapeDtypeStruct((SEG_PAD, D_OUT), jnp.float32),
        ),
        mesh=mesh,
        scratch_types=[
            pltpu.VMEM((CHUNK, D_OUT), jnp.float32),
            pltpu.VMEM((CHUNK, D_OUT), jnp.float32),
            pltpu.VMEM((cpt, CHUNK), jnp.int32),
            pltpu.VMEM_SHARED((SEG_PAD, D_OUT), jnp.float32),
            pltpu.SemaphoreType.DMA,
            pltpu.SemaphoreType.DMA,
        ],
    )
    return f(precision, weighted, idx2d, zeros)


SEG_BLOCK = 2000


def _norm_body(wa_ref, wb_ref, ma_ref, mb_ref, mean_ref, var_ref):
    w = wa_ref[...] + wb_ref[...] + 1e-8
    vi = 1.0 / w
    var_ref[...] = vi
    mean_ref[...] = (ma_ref[...] + mb_ref[...]) * vi


def _normalize(w_a, w_b, m_a, m_b):
    row = lambda i: (i, 0)
    spec = pl.BlockSpec((SEG_BLOCK, D_OUT), row)
    return pl.pallas_call(
        _norm_body,
        grid=(N_SEGMENTS // SEG_BLOCK,),
        in_specs=[spec, spec, spec, spec],
        out_specs=[spec, spec],
        out_shape=[jax.ShapeDtypeStruct((N_SEGMENTS, D_OUT), jnp.float32),
                   jax.ShapeDtypeStruct((N_SEGMENTS, D_OUT), jnp.float32)],
    )(w_a, w_b, m_a, m_b)


ROWS_H = 80000
CHUNKS_H = ROWS_H // CHUNK
CPT_H = 40
IDX_PAD_H = NUM_TILES * CPT_H


def kernel(X, X_idx, W0_mean, W0_logvar, b0_mean, b0_logvar,
           W1_mean, W1_logvar, b1_mean, b1_logvar):
    idx_all = X_idx.reshape(N_CHUNKS, CHUNK)
    pad = jnp.zeros((IDX_PAD_H - CHUNKS_H, CHUNK), jnp.int32)
    idx_a = jnp.concatenate([idx_all[:CHUNKS_H], pad], axis=0)
    idx_b = jnp.concatenate([idx_all[CHUNKS_H:], pad], axis=0)
    zeros = jnp.zeros((SEG_PAD, D_OUT), jnp.float32)

    blocks_a = ROWS_H // ROW_BLOCK
    blocks_b = (N_ROWS - ROWS_H) // ROW_BLOCK
    prec_a, wtd_a = _vb_stage(
        X, 0, blocks_a, W0_mean, W0_logvar, b0_mean, b0_logvar,
        W1_mean, W1_logvar, b1_mean, b1_logvar)
    w_a, m_a = _segment_sums(prec_a, wtd_a, idx_a, zeros)
    prec_b, wtd_b = _vb_stage(
        X, blocks_a, blocks_b, W0_mean, W0_logvar, b0_mean, b0_logvar,
        W1_mean, W1_logvar, b1_mean, b1_logvar)
    w_b, m_b = _segment_sums(prec_b, wtd_b, idx_b, zeros)

    embedd_means, embedd_vars = _normalize(w_a, w_b, m_a, m_b)
    return (embedd_means, embedd_vars)


# speedup vs baseline: 1.3209x; 1.0342x over previous
import functools

import jax
import jax.numpy as jnp
from jax import lax
from jax.experimental import pallas as pl
from jax.experimental.pallas import tpu as pltpu
from jax.experimental.pallas import tpu_sc as plsc

N_ROWS = 160000
N_SEGMENTS = 10000
D_IN = 128
NUM_RF = 256
D_OUT = 128


ROW_BLOCK = 2000

_INV_SQRT_2PI = 0.3989422804014327
_SQRT_2_OVER_PI = 0.7978845608028654


def _vb_body(x_ref, w0m_ref, w0lv_ref, b0m_ref, b0lv_ref,
             w1m_ref, w1lv_ref, b1m_ref, b1lv_ref,
             prec_ref, wtd_ref, w0v_s, w1v_s, w1vpm_s):
    @pl.when(pl.program_id(0) == 0)
    def _():
        w0v_s[...] = jnp.exp(w0lv_ref[...])
        w1v = jnp.exp(w1lv_ref[...])
        w1v_s[...] = w1v
        w1m = w1m_ref[...]
        w1vpm_s[...] = w1v + w1m * w1m

    x = x_ref[...]
    b0m = b0m_ref[...]
    b0v = jnp.exp(b0lv_ref[...])

    out_mean = jnp.dot(x, w0m_ref[...], preferred_element_type=jnp.float32) + b0m
    out_var = jnp.dot(x * x, w0v_s[...], preferred_element_type=jnp.float32) + b0v
    out_var = jnp.maximum(out_var, 1e-8)

    r = lax.rsqrt(out_var)
    a = out_mean * r
    s = out_var * r
    a2 = a * a
    cdf = 0.5 + 0.5 * jnp.tanh(_SQRT_2_OVER_PI * (a * (1.0 + 0.044715 * a2)))
    pdf = jnp.exp(-0.5 * a2) * _INV_SQRT_2PI
    m1 = out_mean * cdf + s * pdf
    v1 = m1 * (out_mean - m1) + out_var * cdf
    v1 = jnp.maximum(v1, 1e-8)

    b1m = b1m_ref[...]
    b1v = jnp.exp(b1lv_ref[...])

    m2 = jnp.dot(m1, w1m_ref[...], preferred_element_type=jnp.float32) + b1m
    v2 = (jnp.dot(v1, w1vpm_s[...], preferred_element_type=jnp.float32)
          + jnp.dot(m1 * m1, w1v_s[...], preferred_element_type=jnp.float32)
          + b1v)
    v2 = jnp.maximum(v2, 1e-8)

    prec = 1.0 / (v2 + 1e-8)
    prec_ref[...] = prec
    wtd_ref[...] = prec * m2


def _vb_stage(X, block_off, n_blocks,
              W0_mean, W0_logvar, b0_mean, b0_logvar,
              W1_mean, W1_logvar, b1_mean, b1_logvar):
    n_rows = n_blocks * ROW_BLOCK
    full = lambda i: (0, 0)
    row = lambda i: (i + block_off, 0)
    out_row = lambda i: (i, 0)
    return pl.pallas_call(
        _vb_body,
        grid=(n_blocks,),
        in_specs=[
            pl.BlockSpec((ROW_BLOCK, D_IN), row),
            pl.BlockSpec((D_IN, NUM_RF), full),
            pl.BlockSpec((D_IN, NUM_RF), full),
            pl.BlockSpec((NUM_RF,), lambda i: (0,)),
            pl.BlockSpec((NUM_RF,), lambda i: (0,)),
            pl.BlockSpec((NUM_RF, D_OUT), full),
            pl.BlockSpec((NUM_RF, D_OUT), full),
            pl.BlockSpec((D_OUT,), lambda i: (0,)),
            pl.BlockSpec((D_OUT,), lambda i: (0,)),
        ],
        out_specs=[
            pl.BlockSpec((ROW_BLOCK, D_OUT), out_row),
            pl.BlockSpec((ROW_BLOCK, D_OUT), out_row),
        ],
        out_shape=[
            jax.ShapeDtypeStruct((n_rows, D_OUT), jnp.float32),
            jax.ShapeDtypeStruct((n_rows, D_OUT), jnp.float32),
        ],
        scratch_shapes=[
            pltpu.VMEM((D_IN, NUM_RF), jnp.float32),
            pltpu.VMEM((NUM_RF, D_OUT), jnp.float32),
            pltpu.VMEM((NUM_RF, D_OUT), jnp.float32),
        ],
    )(X, W0_mean, W0_logvar, b0_mean, b0_logvar,
      W1_mean, W1_logvar, b1_mean, b1_logvar)


NUM_TILES = 16
CHUNK = 128
N_CHUNKS = N_ROWS // CHUNK
SEG_PAD = 10240
SEG_PER_TILE = SEG_PAD // NUM_TILES


def _seg_tile_work(src_hbm, idx_v, acc_sh, buf0, buf1, sem0, sem1, s,
                   n_chunks, cpt):
    base = s * cpt

    def start_load(k, buf, sem):
        @pl.when((k < cpt) & (base + k < n_chunks))
        def _():
            pltpu.async_copy(src_hbm.at[pl.ds((base + k) * CHUNK, CHUNK)],
                             buf, sem)

    def finish(k, buf, sem):
        @pl.when(base + k < n_chunks)
        def _():
            pltpu.make_async_copy(src_hbm.at[pl.ds((base + k) * CHUNK, CHUNK)],
                                  buf, sem).wait()
            pltpu.sync_copy(buf, acc_sh.at[idx_v.at[k]], add=True)

    start_load(0, buf0, sem0)
    start_load(1, buf1, sem1)

    def outer(j, _):
        k = j * 2
        finish(k, buf0, sem0)
        start_load(k + 2, buf0, sem0)
        finish(k + 1, buf1, sem1)
        start_load(k + 3, buf1, sem1)
        return 0
    lax.fori_loop(0, cpt // 2, outer, 0)


def _seg_body(n_chunks, cpt,
              prec_hbm, wtd_hbm, idx_hbm, zeros_hbm,
              wsum_hbm, msum_hbm,
              buf0, buf1, idx_v, acc_sh, sem0, sem1):
    c = lax.axis_index("c")
    s = lax.axis_index("s")
    seg0 = s * SEG_PER_TILE
    pltpu.sync_copy(zeros_hbm.at[pl.ds(seg0, SEG_PER_TILE)],
                    acc_sh.at[pl.ds(seg0, SEG_PER_TILE)])
    pltpu.sync_copy(idx_hbm.at[pl.ds(s * cpt, cpt)], idx_v)
    plsc.subcore_barrier()

    @pl.when(c == 0)
    def _():
        _seg_tile_work(prec_hbm, idx_v, acc_sh, buf0, buf1, sem0, sem1, s,
                       n_chunks, cpt)

    @pl.when(c == 1)
    def _():
        _seg_tile_work(wtd_hbm, idx_v, acc_sh, buf0, buf1, sem0, sem1, s,
                       n_chunks, cpt)

    plsc.subcore_barrier()

    @pl.when(c == 0)
    def _():
        pltpu.sync_copy(acc_sh.at[pl.ds(seg0, SEG_PER_TILE)],
                        wsum_hbm.at[pl.ds(seg0, SEG_PER_TILE)])

    @pl.when(c == 1)
    def _():
        pltpu.sync_copy(acc_sh.at[pl.ds(seg0, SEG_PER_TILE)],
                        msum_hbm.at[pl.ds(seg0, SEG_PER_TILE)])


def _segment_sums(precision, weighted, idx2d, zeros):
    n_chunks = precision.shape[0] // CHUNK
    cpt = idx2d.shape[0] // NUM_TILES
    mesh = plsc.VectorSubcoreMesh(core_axis_name="c", subcore_axis_name="s")
    f = pl.kernel(
        functools.partial(_seg_body, n_chunks, cpt),
        out_type=(
            jax.ShapeDtypeStruct((SEG_PAD, D_OUT), jnp.float32),
            jax.ShapeDtypeStruct((SEG_PAD, D_OUT), jnp.float32),
        ),
        mesh=mesh,
        scratch_types=[
            pltpu.VMEM((CHUNK, D_OUT), jnp.float32),
            pltpu.VMEM((CHUNK, D_OUT), jnp.float32),
            pltpu.VMEM((cpt, CHUNK), jnp.int32),
            pltpu.VMEM_SHARED((SEG_PAD, D_OUT), jnp.float32),
            pltpu.SemaphoreType.DMA,
            pltpu.SemaphoreType.DMA,
        ],
    )
    return f(precision, weighted, idx2d, zeros)


SEG_BLOCK = 2000


def _norm_body(*refs):
    ws = refs[:N_SPLITS]
    ms = refs[N_SPLITS:2 * N_SPLITS]
    mean_ref, var_ref = refs[2 * N_SPLITS:]
    w = ws[0][...]
    for r in ws[1:]:
        w = w + r[...]
    m = ms[0][...]
    for r in ms[1:]:
        m = m + r[...]
    vi = 1.0 / (w + 1e-8)
    var_ref[...] = vi
    mean_ref[...] = m * vi


def _normalize(ws, ms):
    row = lambda i: (i, 0)
    spec = pl.BlockSpec((SEG_BLOCK, D_OUT), row)
    return pl.pallas_call(
        _norm_body,
        grid=(N_SEGMENTS // SEG_BLOCK,),
        in_specs=[spec] * (2 * N_SPLITS),
        out_specs=[spec, spec],
        out_shape=[jax.ShapeDtypeStruct((N_SEGMENTS, D_OUT), jnp.float32),
                   jax.ShapeDtypeStruct((N_SEGMENTS, D_OUT), jnp.float32)],
    )(*ws, *ms)


SPLITS = [(0, 24), (24, 16), (40, 24), (64, 16)]
N_SPLITS = len(SPLITS)


def kernel(X, X_idx, W0_mean, W0_logvar, b0_mean, b0_logvar,
           W1_mean, W1_logvar, b1_mean, b1_logvar):
    idx_all = X_idx.reshape(N_CHUNKS, CHUNK)
    zeros = jnp.zeros((SEG_PAD, D_OUT), jnp.float32)

    ws, ms = [], []
    chunk0 = 0
    for off, nb in SPLITS:
        n_chunks = nb * ROW_BLOCK // CHUNK
        cpt = (-(-n_chunks // NUM_TILES) + 7) // 8 * 8
        pad_rows = NUM_TILES * cpt - n_chunks
        idx_i = idx_all[chunk0:chunk0 + n_chunks]
        if pad_rows:
            idx_i = jnp.concatenate(
                [idx_i, jnp.zeros((pad_rows, CHUNK), jnp.int32)], axis=0)
        prec, wtd = _vb_stage(
            X, off, nb, W0_mean, W0_logvar, b0_mean, b0_logvar,
            W1_mean, W1_logvar, b1_mean, b1_logvar)
        w_i, m_i = _segment_sums(prec, wtd, idx_i, zeros)
        ws.append(w_i)
        ms.append(m_i)
        chunk0 += n_chunks

    embedd_means, embedd_vars = _normalize(ws, ms)
    return (embedd_means, embedd_vars)


# speedup vs baseline: 1.3428x; 1.0166x over previous
import functools

import jax
import jax.numpy as jnp
from jax import lax
from jax.experimental import pallas as pl
from jax.experimental.pallas import tpu as pltpu
from jax.experimental.pallas import tpu_sc as plsc

N_ROWS = 160000
N_SEGMENTS = 10000
D_IN = 128
NUM_RF = 256
D_OUT = 128


ROW_BLOCK = 2000

_INV_SQRT_2PI = 0.3989422804014327
_SQRT_2_OVER_PI = 0.7978845608028654


def _vb_body(x_ref, w0m_ref, w0lv_ref, b0m_ref, b0lv_ref,
             w1m_ref, w1lv_ref, b1m_ref, b1lv_ref,
             prec_ref, wtd_ref, w0v_s, w1v_s, w1vpm_s):
    @pl.when(pl.program_id(0) == 0)
    def _():
        w0v_s[...] = jnp.exp(w0lv_ref[...])
        w1v = jnp.exp(w1lv_ref[...])
        w1v_s[...] = w1v
        w1m = w1m_ref[...]
        w1vpm_s[...] = w1v + w1m * w1m

    x = x_ref[...]
    b0m = b0m_ref[...]
    b0v = jnp.exp(b0lv_ref[...])

    out_mean = jnp.dot(x, w0m_ref[...], preferred_element_type=jnp.float32) + b0m
    out_var = jnp.dot(x * x, w0v_s[...], preferred_element_type=jnp.float32) + b0v
    out_var = jnp.maximum(out_var, 1e-8)

    r = lax.rsqrt(out_var)
    a = out_mean * r
    s = out_var * r
    a2 = a * a
    cdf = 0.5 + 0.5 * jnp.tanh(_SQRT_2_OVER_PI * (a * (1.0 + 0.044715 * a2)))
    pdf = jnp.exp(-0.5 * a2) * _INV_SQRT_2PI
    m1 = out_mean * cdf + s * pdf
    v1 = m1 * (out_mean - m1) + out_var * cdf
    v1 = jnp.maximum(v1, 1e-8)

    b1m = b1m_ref[...]
    b1v = jnp.exp(b1lv_ref[...])

    m2 = jnp.dot(m1, w1m_ref[...], preferred_element_type=jnp.float32) + b1m
    v2 = (jnp.dot(v1, w1vpm_s[...], preferred_element_type=jnp.float32)
          + jnp.dot(m1 * m1, w1v_s[...], preferred_element_type=jnp.float32)
          + b1v)
    v2 = jnp.maximum(v2, 1e-8)

    prec = 1.0 / (v2 + 1e-8)
    prec_ref[...] = prec
    wtd_ref[...] = prec * m2


def _vb_stage(X, block_off, n_blocks,
              W0_mean, W0_logvar, b0_mean, b0_logvar,
              W1_mean, W1_logvar, b1_mean, b1_logvar):
    n_rows = n_blocks * ROW_BLOCK
    full = lambda i: (0, 0)
    row = lambda i: (i + block_off, 0)
    out_row = lambda i: (i, 0)
    return pl.pallas_call(
        _vb_body,
        grid=(n_blocks,),
        in_specs=[
            pl.BlockSpec((ROW_BLOCK, D_IN), row),
            pl.BlockSpec((D_IN, NUM_RF), full),
            pl.BlockSpec((D_IN, NUM_RF), full),
            pl.BlockSpec((NUM_RF,), lambda i: (0,)),
            pl.BlockSpec((NUM_RF,), lambda i: (0,)),
            pl.BlockSpec((NUM_RF, D_OUT), full),
            pl.BlockSpec((NUM_RF, D_OUT), full),
            pl.BlockSpec((D_OUT,), lambda i: (0,)),
            pl.BlockSpec((D_OUT,), lambda i: (0,)),
        ],
        out_specs=[
            pl.BlockSpec((ROW_BLOCK, D_OUT), out_row),
            pl.BlockSpec((ROW_BLOCK, D_OUT), out_row),
        ],
        out_shape=[
            jax.ShapeDtypeStruct((n_rows, D_OUT), jnp.float32),
            jax.ShapeDtypeStruct((n_rows, D_OUT), jnp.float32),
        ],
        scratch_shapes=[
            pltpu.VMEM((D_IN, NUM_RF), jnp.float32),
            pltpu.VMEM((NUM_RF, D_OUT), jnp.float32),
            pltpu.VMEM((NUM_RF, D_OUT), jnp.float32),
        ],
    )(X, W0_mean, W0_logvar, b0_mean, b0_logvar,
      W1_mean, W1_logvar, b1_mean, b1_logvar)


NUM_TILES = 16
CHUNK = 128
N_CHUNKS = N_ROWS // CHUNK
SEG_PAD = 10240
SEG_PER_TILE = SEG_PAD // NUM_TILES


def _seg_tile_work(src_hbm, idx_v, acc_sh, buf0, buf1, sem0, sem1, s,
                   n_chunks, cpt):
    base = s * cpt

    def start_load(k, buf, sem):
        @pl.when((k < cpt) & (base + k < n_chunks))
        def _():
            pltpu.async_copy(src_hbm.at[pl.ds((base + k) * CHUNK, CHUNK)],
                             buf, sem)

    def finish(k, buf, sem):
        @pl.when(base + k < n_chunks)
        def _():
            pltpu.make_async_copy(src_hbm.at[pl.ds((base + k) * CHUNK, CHUNK)],
                                  buf, sem).wait()
            pltpu.sync_copy(buf, acc_sh.at[idx_v.at[k]], add=True)

    start_load(0, buf0, sem0)
    start_load(1, buf1, sem1)

    def outer(j, _):
        k = j * 2
        finish(k, buf0, sem0)
        start_load(k + 2, buf0, sem0)
        finish(k + 1, buf1, sem1)
        start_load(k + 3, buf1, sem1)
        return 0
    lax.fori_loop(0, cpt // 2, outer, 0)


def _seg_body(n_chunks, cpt,
              prec_hbm, wtd_hbm, idx_hbm, init_w_hbm, init_m_hbm,
              wsum_hbm, msum_hbm,
              buf0, buf1, idx_v, acc_sh, sem0, sem1):
    c = lax.axis_index("c")
    s = lax.axis_index("s")
    seg0 = s * SEG_PER_TILE
    @pl.when(c == 0)
    def _():
        pltpu.sync_copy(init_w_hbm.at[pl.ds(seg0, SEG_PER_TILE)],
                        acc_sh.at[pl.ds(seg0, SEG_PER_TILE)])

    @pl.when(c == 1)
    def _():
        pltpu.sync_copy(init_m_hbm.at[pl.ds(seg0, SEG_PER_TILE)],
                        acc_sh.at[pl.ds(seg0, SEG_PER_TILE)])

    pltpu.sync_copy(idx_hbm.at[pl.ds(s * cpt, cpt)], idx_v)
    plsc.subcore_barrier()

    @pl.when(c == 0)
    def _():
        _seg_tile_work(prec_hbm, idx_v, acc_sh, buf0, buf1, sem0, sem1, s,
                       n_chunks, cpt)

    @pl.when(c == 1)
    def _():
        _seg_tile_work(wtd_hbm, idx_v, acc_sh, buf0, buf1, sem0, sem1, s,
                       n_chunks, cpt)

    plsc.subcore_barrier()

    @pl.when(c == 0)
    def _():
        pltpu.sync_copy(acc_sh.at[pl.ds(seg0, SEG_PER_TILE)],
                        wsum_hbm.at[pl.ds(seg0, SEG_PER_TILE)])

    @pl.when(c == 1)
    def _():
        pltpu.sync_copy(acc_sh.at[pl.ds(seg0, SEG_PER_TILE)],
                        msum_hbm.at[pl.ds(seg0, SEG_PER_TILE)])


def _segment_sums(precision, weighted, idx2d, init_w, init_m):
    n_chunks = precision.shape[0] // CHUNK
    cpt = idx2d.shape[0] // NUM_TILES
    mesh = plsc.VectorSubcoreMesh(core_axis_name="c", subcore_axis_name="s")
    f = pl.kernel(
        functools.partial(_seg_body, n_chunks, cpt),
        out_type=(
            jax.ShapeDtypeStruct((SEG_PAD, D_OUT), jnp.float32),
            jax.ShapeDtypeStruct((SEG_PAD, D_OUT), jnp.float32),
        ),
        mesh=mesh,
        scratch_types=[
            pltpu.VMEM((CHUNK, D_OUT), jnp.float32),
            pltpu.VMEM((CHUNK, D_OUT), jnp.float32),
            pltpu.VMEM((cpt, CHUNK), jnp.int32),
            pltpu.VMEM_SHARED((SEG_PAD, D_OUT), jnp.float32),
            pltpu.SemaphoreType.DMA,
            pltpu.SemaphoreType.DMA,
        ],
    )
    return f(precision, weighted, idx2d, init_w, init_m)


SEG_BLOCK = 2000


def _norm_body(w_ref, m_ref, mean_ref, var_ref):
    vi = 1.0 / (w_ref[...] + 1e-8)
    var_ref[...] = vi
    mean_ref[...] = m_ref[...] * vi


def _normalize(w_sum, m_sum):
    row = lambda i: (i, 0)
    spec = pl.BlockSpec((SEG_BLOCK, D_OUT), row)
    return pl.pallas_call(
        _norm_body,
        grid=(N_SEGMENTS // SEG_BLOCK,),
        in_specs=[spec, spec],
        out_specs=[spec, spec],
        out_shape=[jax.ShapeDtypeStruct((N_SEGMENTS, D_OUT), jnp.float32),
                   jax.ShapeDtypeStruct((N_SEGMENTS, D_OUT), jnp.float32)],
    )(w_sum, m_sum)


SPLITS = [(0, 24), (24, 16), (40, 24), (64, 16)]
N_SPLITS = len(SPLITS)


def kernel(X, X_idx, W0_mean, W0_logvar, b0_mean, b0_logvar,
           W1_mean, W1_logvar, b1_mean, b1_logvar):
    idx_all = X_idx.reshape(N_CHUNKS, CHUNK)
    zeros = jnp.zeros((SEG_PAD, D_OUT), jnp.float32)

    w_acc, m_acc = zeros, zeros
    chunk0 = 0
    for off, nb in SPLITS:
        n_chunks = nb * ROW_BLOCK // CHUNK
        cpt = (-(-n_chunks // NUM_TILES) + 7) // 8 * 8
        pad_rows = NUM_TILES * cpt - n_chunks
        idx_i = idx_all[chunk0:chunk0 + n_chunks]
        if pad_rows:
            idx_i = jnp.concatenate(
                [idx_i, jnp.zeros((pad_rows, CHUNK), jnp.int32)], axis=0)
        prec, wtd = _vb_stage(
            X, off, nb, W0_mean, W0_logvar, b0_mean, b0_logvar,
            W1_mean, W1_logvar, b1_mean, b1_logvar)
        w_acc, m_acc = _segment_sums(prec, wtd, idx_i, w_acc, m_acc)
        chunk0 += n_chunks

    embedd_means, embedd_vars = _normalize(w_acc, m_acc)
    return (embedd_means, embedd_vars)
